# Initial kernel scaffold; baseline (speedup 1.0000x reference)
import functools

import jax
import jax.numpy as jnp
from jax import lax
from jax.experimental import pallas as pl
from jax.experimental.pallas import tpu as pltpu
from jax.experimental.pallas import tpu_sc as plsc

_NC = 2
_NS = 16
_NW = _NC * _NS
_L = 16

_SELU_SCALE = 1.0507009873554805
_SELU_ALPHA = 1.6732632423543772


def _sc_partials(x, src, dst, w, n, d):
    e = src.size
    epw = e // _NW
    b = 80
    nb = epw // b
    rpt = n // _NS
    zr = 125
    nz = rpt // zr

    src3 = src.reshape(_NW, nb, b)
    dst3 = dst.reshape(_NW, nb, b)
    w3 = w.reshape(_NW, nb, b)

    mesh = plsc.VectorSubcoreMesh(core_axis_name="c", subcore_axis_name="s")

    @functools.partial(
        pl.kernel,
        mesh=mesh,
        out_type=jax.ShapeDtypeStruct((_NC, n, d), jnp.float32),
        scratch_types=[
            pltpu.VMEM((nb, b), jnp.int32),
            pltpu.VMEM((nb, b), jnp.int32),
            pltpu.VMEM((nb, b), jnp.float32),
            pltpu.VMEM((b, d), jnp.float32),
            pltpu.VMEM((zr, d), jnp.float32),
            pltpu.VMEM_SHARED((n, d), jnp.float32),
            pltpu.SemaphoreType.DMA,
        ],
    )
    def scatter_kernel(x_hbm, src_hbm, dst_hbm, w_hbm, out_hbm,
                       src_v, dst_v, w_v, rows_v, zero_v, agg_sh, sem):
        cid = lax.axis_index("c")
        sid = lax.axis_index("s")
        wid = sid * _NC + cid

        pltpu.sync_copy(src_hbm.at[wid], src_v)
        pltpu.sync_copy(dst_hbm.at[wid], dst_v)
        pltpu.sync_copy(w_hbm.at[wid], w_v)

        def zstore(i, carry):
            for q in range(d // _L):
                zero_v[i, pl.ds(q * _L, _L)] = jnp.zeros((_L,), jnp.float32)
            return carry
        lax.fori_loop(0, zr, zstore, 0)
        for q in range(nz):
            pltpu.sync_copy(zero_v, agg_sh.at[pl.ds(sid * rpt + q * zr, zr)])
        plsc.subcore_barrier()

        def block(j, carry):
            pltpu.async_copy(x_hbm.at[src_v.at[j]], rows_v, sem).wait()
            jj = jnp.full((_L,), j, jnp.int32)

            def row(i, rcarry):
                ws = plsc.load_gather(w_v, [jj, jnp.full((_L,), i, jnp.int32)])
                for q in range(d // _L):
                    rows_v[i, pl.ds(q * _L, _L)] = rows_v[i, pl.ds(q * _L, _L)] * ws
                return rcarry
            lax.fori_loop(0, b, row, 0)

            pltpu.sync_copy(rows_v, agg_sh.at[dst_v.at[j]], add=True)
            return carry
        lax.fori_loop(0, nb, block, 0)

        plsc.subcore_barrier()
        for q in range(nz):
            rows = pl.ds(sid * rpt + q * zr, zr)
            pltpu.sync_copy(agg_sh.at[rows], out_hbm.at[cid].at[rows])

    return scatter_kernel(x, src3, dst3, w3)


def _epilogue(x, partials, kmat, bias2, skip2, n, d, c):
    rb = 1000

    def body(x_ref, p_ref, k_ref, b_ref, s_ref, o_ref):
        km = k_ref[...]
        acc = jnp.dot(x_ref[...], km * s_ref[...], preferred_element_type=jnp.float32)
        acc = acc + jnp.dot(p_ref[0] + p_ref[1], km, preferred_element_type=jnp.float32)
        acc = acc + b_ref[...]
        neg = _SELU_ALPHA * (jnp.exp(jnp.minimum(acc, 0.0)) - 1.0)
        o_ref[...] = _SELU_SCALE * jnp.where(acc > 0.0, acc, neg)

    return pl.pallas_call(
        body,
        grid=(n // rb,),
        in_specs=[
            pl.BlockSpec((rb, d), lambda i: (i, 0)),
            pl.BlockSpec((_NC, rb, c), lambda i: (0, i, 0)),
            pl.BlockSpec((d, c), lambda i: (0, 0)),
            pl.BlockSpec((1, c), lambda i: (0, 0)),
            pl.BlockSpec((1, c), lambda i: (0, 0)),
        ],
        out_specs=pl.BlockSpec((rb, c), lambda i: (i, 0)),
        out_shape=jax.ShapeDtypeStruct((n, c), jnp.float32),
    )(x, partials, kmat, bias2, skip2)


def kernel(features, edge_index, edge_weight, kernel, bias, skip_weight):
    n, d = features.shape
    c = kernel.shape[1]
    dst = edge_index[0]
    src = edge_index[1]
    partials = _sc_partials(features, src, dst, edge_weight, n, d)
    return _epilogue(features, partials, kernel,
                     bias.reshape(1, c), skip_weight.reshape(1, c), n, d, c)


# speedup vs baseline: 7.2008x; 7.2008x over previous
import functools

import jax
import jax.numpy as jnp
from jax import lax
from jax.experimental import pallas as pl
from jax.experimental.pallas import tpu as pltpu
from jax.experimental.pallas import tpu_sc as plsc

_NC = 2
_NS = 16
_NW = _NC * _NS
_L = 16

_SELU_SCALE = 1.0507009873554805
_SELU_ALPHA = 1.6732632423543772


def _sc_partials(x, src, dst, w, n, d):
    e = src.size
    b = 128
    sbb = 8
    epw = -(-e // (_NW * b * sbb)) * b * sbb
    ep = epw * _NW
    pad = ep - e
    nb = epw // b
    nsb = nb // sbb

    idx_pad = jnp.arange(pad, dtype=jnp.int32) % n
    src_p = jnp.concatenate([src, idx_pad]).reshape(_NW, nb, b)
    dst_p = jnp.concatenate([dst, idx_pad]).reshape(_NW, nb, b)
    w_p = jnp.concatenate([w, jnp.zeros((pad,), jnp.float32)]).reshape(_NW, nb, b)

    npad = -(-n // (_NS * 128)) * _NS * 128
    rpt = npad // _NS
    nz = rpt // b

    mesh = plsc.VectorSubcoreMesh(core_axis_name="c", subcore_axis_name="s")

    @functools.partial(
        pl.kernel,
        mesh=mesh,
        out_type=jax.ShapeDtypeStruct((_NC, npad, d), jnp.float32),
        scratch_types=[
            pltpu.VMEM((sbb, b), jnp.int32),
            pltpu.VMEM((sbb, b), jnp.int32),
            pltpu.VMEM((sbb, b), jnp.float32),
            pltpu.VMEM((b, d), jnp.float32),
            pltpu.VMEM_SHARED((npad, d), jnp.float32),
            pltpu.SemaphoreType.DMA,
        ],
    )
    def scatter_kernel(x_hbm, src_hbm, dst_hbm, w_hbm, out_hbm,
                       src_v, dst_v, w_v, rows_v, agg_sh, sem):
        cid = lax.axis_index("c")
        sid = lax.axis_index("s")
        wid = sid * _NC + cid

        def zstore(i, carry):
            for q in range(d // _L):
                rows_v[i, pl.ds(q * _L, _L)] = jnp.zeros((_L,), jnp.float32)
            return carry
        lax.fori_loop(0, b, zstore, 0)
        for q in range(nz):
            pltpu.sync_copy(rows_v, agg_sh.at[pl.ds(sid * rpt + q * b, b)])
        plsc.subcore_barrier()

        def super_block(sb, carry):
            grp = pl.ds(sb * sbb, sbb)
            pltpu.sync_copy(src_hbm.at[wid].at[grp], src_v)
            pltpu.sync_copy(dst_hbm.at[wid].at[grp], dst_v)
            pltpu.sync_copy(w_hbm.at[wid].at[grp], w_v)

            def block(j, bcarry):
                pltpu.async_copy(x_hbm.at[src_v.at[j]], rows_v, sem).wait()

                def row16(i16, rcarry):
                    wv = w_v[j, pl.ds(i16 * _L, _L)]
                    for k in range(_L):
                        ws = jnp.full((_L,), wv[k], jnp.float32)
                        r = i16 * _L + k
                        for q in range(d // _L):
                            rows_v[r, pl.ds(q * _L, _L)] = (
                                rows_v[r, pl.ds(q * _L, _L)] * ws)
                    return rcarry
                lax.fori_loop(0, b // _L, row16, 0)

                pltpu.sync_copy(rows_v, agg_sh.at[dst_v.at[j]], add=True)
                return bcarry
            lax.fori_loop(0, sbb, block, 0)
            return carry
        lax.fori_loop(0, nsb, super_block, 0)

        plsc.subcore_barrier()
        for q in range(nz):
            rows = pl.ds(sid * rpt + q * b, b)
            pltpu.sync_copy(agg_sh.at[rows], out_hbm.at[cid].at[rows])

    return scatter_kernel(x, src_p, dst_p, w_p)


def _epilogue(x, partials, kmat, bias2, skip2, n, d, c):
    rb = 1000

    def body(x_ref, p_ref, k_ref, b_ref, s_ref, o_ref):
        km = k_ref[...]
        acc = jnp.dot(x_ref[...], km * s_ref[...], preferred_element_type=jnp.float32)
        acc = acc + jnp.dot(p_ref[0] + p_ref[1], km, preferred_element_type=jnp.float32)
        acc = acc + b_ref[...]
        neg = _SELU_ALPHA * (jnp.exp(jnp.minimum(acc, 0.0)) - 1.0)
        o_ref[...] = _SELU_SCALE * jnp.where(acc > 0.0, acc, neg)

    return pl.pallas_call(
        body,
        grid=(n // rb,),
        in_specs=[
            pl.BlockSpec((rb, d), lambda i: (i, 0)),
            pl.BlockSpec((_NC, rb, c), lambda i: (0, i, 0)),
            pl.BlockSpec((d, c), lambda i: (0, 0)),
            pl.BlockSpec((1, c), lambda i: (0, 0)),
            pl.BlockSpec((1, c), lambda i: (0, 0)),
        ],
        out_specs=pl.BlockSpec((rb, c), lambda i: (i, 0)),
        out_shape=jax.ShapeDtypeStruct((n, c), jnp.float32),
    )(x, partials, kmat, bias2, skip2)


def kernel(features, edge_index, edge_weight, kernel, bias, skip_weight):
    n, d = features.shape
    c = kernel.shape[1]
    dst = edge_index[0]
    src = edge_index[1]
    partials = _sc_partials(features, src, dst, edge_weight, n, d)
    return _epilogue(features, partials, kernel,
                     bias.reshape(1, c), skip_weight.reshape(1, c), n, d, c)
